# baseline (speedup 1.0000x reference)
import functools

import jax
import jax.numpy as jnp
from jax.experimental import pallas as pl

_RANGE_MIN = 24
_RANGE_MAX = 31
_IN_DIM = 128
_EMB_DIM = 18
_OUT_W = _IN_DIM * _EMB_DIM


def _body(x_ref, m_ref, o_ref):
    v = x_ref[...]
    rep = jnp.dot(v, m_ref[...], preferred_element_type=jnp.float32)
    j = jax.lax.broadcasted_iota(jnp.int32, rep.shape, 1)
    e = j % _EMB_DIM
    is_card = (j >= _RANGE_MIN * _EMB_DIM) & (j < _RANGE_MAX * _EMB_DIM)
    vi = jnp.floor(rep)
    r = jnp.floor(vi * 0.25)
    s = vi - 4.0 * r
    ef = e.astype(jnp.float32)
    one = jnp.ones_like(rep)
    zero = jnp.zeros_like(rep)
    rank_oh = jnp.where(r == ef, one, zero)
    suit_oh = jnp.where(s == ef - 13.0, one, zero)
    card_val = jnp.where(e < 13, rank_oh, jnp.where(e < 17, suit_oh, one))
    o_ref[...] = jnp.where(is_card, card_val, rep)


@jax.jit
def _run(x2):
    b = x2.shape[0]
    bb = 512
    cols = jnp.arange(_OUT_W, dtype=jnp.int32) // _EMB_DIM
    m = (jnp.arange(_IN_DIM, dtype=jnp.int32)[:, None] == cols[None, :]).astype(
        jnp.float32
    )
    out = pl.pallas_call(
        _body,
        grid=(b // bb,),
        in_specs=[
            pl.BlockSpec((bb, _IN_DIM), lambda i: (i, 0)),
            pl.BlockSpec((_IN_DIM, _OUT_W), lambda i: (0, 0)),
        ],
        out_specs=pl.BlockSpec((bb, _OUT_W), lambda i: (i, 0)),
        out_shape=jax.ShapeDtypeStruct((b, _OUT_W), jnp.float32),
    )(x2, m)
    return out.reshape(b, _IN_DIM, _EMB_DIM)


def kernel(x):
    if x.ndim == 3:
        x = x[:, 0, :]
    return _run(x)


# speedup vs baseline: 1.0042x; 1.0042x over previous
import functools

import jax
import jax.numpy as jnp
from jax.experimental import pallas as pl

_RANGE_MIN = 24
_RANGE_MAX = 31
_IN_DIM = 128
_EMB_DIM = 18
_OUT_W = _IN_DIM * _EMB_DIM


_LO = 384
_HI = 640


def _body(x_ref, m_ref, o_ref):
    v = x_ref[...].astype(jnp.bfloat16)
    rep = jnp.dot(v, m_ref[...], preferred_element_type=jnp.float32)
    stripe = rep[:, _LO:_HI]
    j = jax.lax.broadcasted_iota(jnp.int32, stripe.shape, 1) + _LO
    e = j % _EMB_DIM
    is_card = (j >= _RANGE_MIN * _EMB_DIM) & (j < _RANGE_MAX * _EMB_DIM)
    vi = jnp.floor(stripe)
    r = jnp.floor(vi * 0.25)
    s = vi - 4.0 * r
    ef = e.astype(jnp.float32)
    one = jnp.ones_like(stripe)
    zero = jnp.zeros_like(stripe)
    rank_oh = jnp.where(r == ef, one, zero)
    suit_oh = jnp.where(s == ef - 13.0, one, zero)
    card_val = jnp.where(e < 13, rank_oh, jnp.where(e < 17, suit_oh, one))
    o_ref[:, :_LO] = rep[:, :_LO]
    o_ref[:, _LO:_HI] = jnp.where(is_card, card_val, stripe)
    o_ref[:, _HI:] = rep[:, _HI:]


@jax.jit
def _run(x2):
    b = x2.shape[0]
    bb = 512
    cols = jnp.arange(_OUT_W, dtype=jnp.int32) // _EMB_DIM
    m = (jnp.arange(_IN_DIM, dtype=jnp.int32)[:, None] == cols[None, :]).astype(
        jnp.bfloat16
    )
    out = pl.pallas_call(
        _body,
        grid=(b // bb,),
        in_specs=[
            pl.BlockSpec((bb, _IN_DIM), lambda i: (i, 0)),
            pl.BlockSpec((_IN_DIM, _OUT_W), lambda i: (0, 0)),
        ],
        out_specs=pl.BlockSpec((bb, _OUT_W), lambda i: (i, 0)),
        out_shape=jax.ShapeDtypeStruct((b, _OUT_W), jnp.float32),
    )(x2, m)
    return out.reshape(b, _IN_DIM, _EMB_DIM)


def kernel(x):
    if x.ndim == 3:
        x = x[:, 0, :]
    return _run(x)


# speedup vs baseline: 2.1048x; 2.0959x over previous
import jax
import jax.numpy as jnp
from jax.experimental import pallas as pl

_RANGE_MIN = 24
_RANGE_MAX = 31
_IN_DIM = 128
_EMB_DIM = 18


def _body(x_ref, o_ref):
    v = x_ref[...]
    bb = v.shape[0]
    shape = (bb, _EMB_DIM, _IN_DIM)
    rep = jnp.broadcast_to(v[:, None, :], shape)
    c = jax.lax.broadcasted_iota(jnp.int32, shape, 2)
    e = jax.lax.broadcasted_iota(jnp.int32, shape, 1)
    is_card = (c >= _RANGE_MIN) & (c < _RANGE_MAX)
    vi = jnp.floor(rep)
    r = jnp.floor(vi * 0.25)
    s = vi - 4.0 * r
    ef = e.astype(jnp.float32)
    one = jnp.ones(shape, jnp.float32)
    zero = jnp.zeros(shape, jnp.float32)
    rank_oh = jnp.where(r == ef, one, zero)
    suit_oh = jnp.where(s == ef - 13.0, one, zero)
    card_val = jnp.where(e < 13, rank_oh, jnp.where(e < 17, suit_oh, one))
    o_ref[...] = jnp.where(is_card, card_val, rep)


@jax.jit
def _run(x2):
    b = x2.shape[0]
    bb = 256
    out = pl.pallas_call(
        _body,
        grid=(b // bb,),
        in_specs=[pl.BlockSpec((bb, _IN_DIM), lambda i: (i, 0))],
        out_specs=pl.BlockSpec((bb, _EMB_DIM, _IN_DIM), lambda i: (i, 0, 0)),
        out_shape=jax.ShapeDtypeStruct((b, _EMB_DIM, _IN_DIM), jnp.float32),
    )(x2)
    return out.transpose(0, 2, 1)


def kernel(x):
    if x.ndim == 3:
        x = x[:, 0, :]
    return _run(x)


# speedup vs baseline: 2.2659x; 1.0765x over previous
import jax
import jax.numpy as jnp
from jax.experimental import pallas as pl

_RANGE_MIN = 24
_RANGE_MAX = 31
_IN_DIM = 128
_EMB_DIM = 18


def _body(x_ref, o_ref):
    v = x_ref[...]
    bb = v.shape[0]
    shape = (bb, _EMB_DIM, _IN_DIM)
    rep = jnp.broadcast_to(v[:, None, :], shape)
    c = jax.lax.broadcasted_iota(jnp.int32, shape, 2)
    e = jax.lax.broadcasted_iota(jnp.int32, shape, 1)
    is_card = (c >= _RANGE_MIN) & (c < _RANGE_MAX)
    vi = jnp.floor(rep)
    r = jnp.floor(vi * 0.25)
    s = vi - 4.0 * r
    ef = e.astype(jnp.float32)
    one = jnp.ones(shape, jnp.float32)
    zero = jnp.zeros(shape, jnp.float32)
    rank_oh = jnp.where(r == ef, one, zero)
    suit_oh = jnp.where(s == ef - 13.0, one, zero)
    card_val = jnp.where(e < 13, rank_oh, jnp.where(e < 17, suit_oh, one))
    o_ref[...] = jnp.where(is_card, card_val, rep)


@jax.jit
def _run(x2):
    b = x2.shape[0]
    bb = 512
    out = pl.pallas_call(
        _body,
        grid=(b // bb,),
        in_specs=[pl.BlockSpec((bb, _IN_DIM), lambda i: (i, 0))],
        out_specs=pl.BlockSpec((bb, _EMB_DIM, _IN_DIM), lambda i: (i, 0, 0)),
        out_shape=jax.ShapeDtypeStruct((b, _EMB_DIM, _IN_DIM), jnp.float32),
    )(x2)
    return out.transpose(0, 2, 1)


def kernel(x):
    if x.ndim == 3:
        x = x[:, 0, :]
    return _run(x)


# speedup vs baseline: 2.2846x; 1.0083x over previous
import jax
import jax.numpy as jnp
from jax.experimental import pallas as pl

_RANGE_MIN = 24
_RANGE_MAX = 31
_IN_DIM = 128
_EMB_DIM = 18


def _body(x_ref, o_ref):
    v = x_ref[...]
    bb = v.shape[0]
    shape = (bb, _EMB_DIM, _IN_DIM)
    rep = jnp.broadcast_to(v[:, None, :], shape)
    c = jax.lax.broadcasted_iota(jnp.int32, shape, 2)
    e = jax.lax.broadcasted_iota(jnp.int32, shape, 1)
    is_card = (c >= _RANGE_MIN) & (c < _RANGE_MAX)
    vi = jnp.floor(rep)
    r = jnp.floor(vi * 0.25)
    s = vi - 4.0 * r
    ef = e.astype(jnp.float32)
    one = jnp.ones(shape, jnp.float32)
    zero = jnp.zeros(shape, jnp.float32)
    rank_oh = jnp.where(r == ef, one, zero)
    suit_oh = jnp.where(s == ef - 13.0, one, zero)
    card_val = jnp.where(e < 13, rank_oh, jnp.where(e < 17, suit_oh, one))
    o_ref[...] = jnp.where(is_card, card_val, rep)


@jax.jit
def _run(x2):
    b = x2.shape[0]
    bb = 1024
    out = pl.pallas_call(
        _body,
        grid=(b // bb,),
        in_specs=[pl.BlockSpec((bb, _IN_DIM), lambda i: (i, 0))],
        out_specs=pl.BlockSpec((bb, _EMB_DIM, _IN_DIM), lambda i: (i, 0, 0)),
        out_shape=jax.ShapeDtypeStruct((b, _EMB_DIM, _IN_DIM), jnp.float32),
    )(x2)
    return out.transpose(0, 2, 1)


def kernel(x):
    if x.ndim == 3:
        x = x[:, 0, :]
    return _run(x)
